# baseline (speedup 1.0000x reference)
import functools

import jax
import jax.numpy as jnp
from jax import lax
from jax.experimental import pallas as pl
from jax.experimental.pallas import tpu as pltpu
from jax.experimental.pallas import tpu_sc as plsc

BATCH = 4
WINDOW = 8192
D_MODEL = 1024
SC_BATCH = 2
NUM_CORES = 2
NUM_SUBCORES = 16
NUM_WORKERS = NUM_CORES * NUM_SUBCORES
ROWS_PER_WORKER = WINDOW // NUM_WORKERS
R = 8
STEPS = ROWS_PER_WORKER // R
NBUF = 3
LANES = 16
CHUNKS = D_MODEL // LANES

TC_ROWS = 256


def _sc_body(x_hbm, t_hbm, out_hbm, buf, tbuf, in_sem, out_sem):
    wid = lax.axis_index("s") * NUM_CORES + lax.axis_index("c")
    base = wid * ROWS_PER_WORKER

    def start_in(s, slot):
        w0 = base + s * R
        return [
            pltpu.async_copy(t_hbm.at[pl.ds(w0, R)], tbuf.at[slot],
                             in_sem.at[slot]),
            pltpu.async_copy(x_hbm.at[:SC_BATCH, pl.ds(w0, R)], buf.at[slot],
                             in_sem.at[slot]),
        ]

    def start_out(s, slot):
        w0 = base + s * R
        return [pltpu.async_copy(buf.at[slot],
                                 out_hbm.at[:, pl.ds(w0, R)],
                                 out_sem.at[slot])]

    def compute(slot):
        def chunk(c, carry):
            o = c * LANES
            for r in range(R):
                t = tbuf[slot, r, pl.ds(o, LANES)]
                for b in range(SC_BATCH):
                    plsc.addupdate(buf.at[slot, b, r, pl.ds(o, LANES)], t)
            return carry

        lax.fori_loop(0, CHUNKS, chunk, 0)

    in_h = {0: start_in(0, 0)}
    out_h = {}
    for s in range(STEPS):
        slot = s % NBUF
        if s + 1 < STEPS:
            if s - 2 >= 0:
                for h in out_h[s - 2]:
                    h.wait()
            in_h[s + 1] = start_in(s + 1, (s + 1) % NBUF)
        for h in in_h[s]:
            h.wait()
        compute(slot)
        out_h[s] = start_out(s, slot)
    for s in (STEPS - 2, STEPS - 1):
        for h in out_h[s]:
            h.wait()


def _tc_body(x_ref, t_ref, o_ref):
    o_ref[...] = x_ref[...] + t_ref[...][None, :, :]


@jax.jit
def kernel(x, pos_table):
    mesh = plsc.VectorSubcoreMesh(core_axis_name="c", subcore_axis_name="s")
    sc_fn = functools.partial(
        pl.kernel,
        mesh=mesh,
        out_type=jax.ShapeDtypeStruct((SC_BATCH, WINDOW, D_MODEL),
                                      jnp.float32),
        scratch_types=[
            pltpu.VMEM((NBUF, SC_BATCH, R, D_MODEL), jnp.float32),
            pltpu.VMEM((NBUF, R, D_MODEL), jnp.float32),
            pltpu.SemaphoreType.DMA((NBUF,)),
            pltpu.SemaphoreType.DMA((NBUF,)),
        ],
    )(_sc_body)
    sc_out = sc_fn(x, pos_table)

    tc_batch = BATCH - SC_BATCH
    tc_out = pl.pallas_call(
        _tc_body,
        grid=(WINDOW // TC_ROWS,),
        in_specs=[
            pl.BlockSpec((tc_batch, TC_ROWS, D_MODEL),
                         lambda i: (SC_BATCH // tc_batch, i, 0)),
            pl.BlockSpec((TC_ROWS, D_MODEL), lambda i: (i, 0)),
        ],
        out_specs=pl.BlockSpec((tc_batch, TC_ROWS, D_MODEL),
                               lambda i: (0, i, 0)),
        out_shape=jax.ShapeDtypeStruct((tc_batch, WINDOW, D_MODEL),
                                       jnp.float32),
    )(x, pos_table)

    return jnp.concatenate([sc_out, tc_out], axis=0)


# speedup vs baseline: 1.6021x; 1.6021x over previous
import functools

import jax
import jax.numpy as jnp
from jax import lax
from jax.experimental import pallas as pl
from jax.experimental.pallas import tpu as pltpu
from jax.experimental.pallas import tpu_sc as plsc

BATCH = 4
WINDOW = 8192
D_MODEL = 1024
NUM_CORES = 2
NUM_SUBCORES = 16
NUM_WORKERS = NUM_CORES * NUM_SUBCORES
ROWS_PER_WORKER = WINDOW // NUM_WORKERS
R = 4
STEPS = ROWS_PER_WORKER // R
NBUF = 4
LANES = 16
CHUNKS = D_MODEL // LANES


def _body(x_hbm, t_hbm, out_hbm, buf, tbuf, in_sem, out_sem):
    wid = lax.axis_index("s") * NUM_CORES + lax.axis_index("c")
    base = wid * ROWS_PER_WORKER

    def in_copies(g, slot):
        w0 = base + g * R
        return [
            pltpu.make_async_copy(t_hbm.at[pl.ds(w0, R)], tbuf.at[slot],
                                  in_sem.at[slot]),
            pltpu.make_async_copy(x_hbm.at[:, pl.ds(w0, R)], buf.at[slot],
                                  in_sem.at[slot]),
        ]

    def out_copies(g, slot):
        w0 = base + g * R
        return [pltpu.make_async_copy(buf.at[slot],
                                      out_hbm.at[:, pl.ds(w0, R)],
                                      out_sem.at[slot])]

    def start(copies):
        for c in copies:
            c.start()

    def wait(copies):
        for c in copies:
            c.wait()

    UNROLL = 4

    def compute(slot):
        def chunk(c, carry):
            o0 = c * (UNROLL * LANES)
            for u in range(UNROLL):
                o = o0 + u * LANES
                for r in range(R):
                    t = tbuf[slot, r, pl.ds(o, LANES)]
                    for b in range(BATCH):
                        plsc.addupdate(buf.at[slot, b, r, pl.ds(o, LANES)], t)
            return carry

        lax.fori_loop(0, CHUNKS // UNROLL, chunk, 0)

    start(in_copies(0, 0))

    def superstep(it, carry):
        g0 = it * NBUF
        for j in range(NBUF):
            g = g0 + j
            nxt_slot = (j + 1) % NBUF

            @pl.when(g + 1 < STEPS)
            def _():
                @pl.when(g >= NBUF - 1)
                def _():
                    wait(out_copies(g - (NBUF - 1), nxt_slot))

                start(in_copies(g + 1, nxt_slot))

            wait(in_copies(g, j))
            compute(j)
            start(out_copies(g, j))
        return carry

    lax.fori_loop(0, STEPS // NBUF, superstep, 0)
    for g in range(STEPS - NBUF, STEPS):
        wait(out_copies(g, g % NBUF))


@jax.jit
def kernel(x, pos_table):
    mesh = plsc.VectorSubcoreMesh(core_axis_name="c", subcore_axis_name="s")
    f = functools.partial(
        pl.kernel,
        mesh=mesh,
        out_type=jax.ShapeDtypeStruct((BATCH, WINDOW, D_MODEL), jnp.float32),
        scratch_types=[
            pltpu.VMEM((NBUF, BATCH, R, D_MODEL), jnp.float32),
            pltpu.VMEM((NBUF, R, D_MODEL), jnp.float32),
            pltpu.SemaphoreType.DMA((NBUF,)),
            pltpu.SemaphoreType.DMA((NBUF,)),
        ],
    )(_body)
    return f(x, pos_table)
